# baseline (speedup 1.0000x reference)
import jax
import jax.numpy as jnp
from jax.experimental import pallas as pl
from jax.experimental.pallas import tpu as pltpu

_N = 8192
_K = 4096
_B = 128
_R = 8
_C = 1024
_ROWS = 16


def _count_ge(u, cand):
    return jnp.sum(jnp.where(u >= cand, jnp.int32(1), jnp.int32(0)))


def _body(w_ref, x_ref, mask_ref, o_ref, mvec_ref):
    step = pl.program_id(0)

    @pl.when(step == 0)
    def _select():
        v = jnp.abs(w_ref[...])
        u = jax.lax.bitcast_convert_type(v, jnp.int32)

        t = jnp.int32(0)
        for b, hi in ((28, 7), (24, 15), (20, 15), (16, 15),
                      (12, 15), (8, 15), (4, 15), (0, 15)):
            d = jnp.int32(0)
            for j in range(1, hi + 1):
                cnt = _count_ge(u, t | jnp.int32(j << b))
                d = d + jnp.where(cnt >= _K, jnp.int32(1), jnp.int32(0))
            t = t | jax.lax.shift_left(d, b)

        gt = u > t
        eq = u == t
        n_gt = jnp.sum(jnp.where(gt, jnp.int32(1), jnp.int32(0)))
        ties = _K - n_gt

        fidx = (jax.lax.broadcasted_iota(jnp.int32, (_R, _C), 0) * _C
                + jax.lax.broadcasted_iota(jnp.int32, (_R, _C), 1))
        m = jnp.int32(0)
        for b in (12, 8, 4, 0):
            d = jnp.int32(0)
            for j in range(1, 16):
                cand = m | jnp.int32(j << b)
                cnt = jnp.sum(jnp.where(eq & (fidx < cand),
                                        jnp.int32(1), jnp.int32(0)))
                d = d + jnp.where(cnt <= ties - 1, jnp.int32(1), jnp.int32(0))
            m = m | jax.lax.shift_left(d, b)

        keep = gt | (eq & (fidx <= m))
        maskv = jnp.where(keep, jnp.float32(1.0), jnp.float32(0.0))
        mvec_ref[...] = maskv
        mask_ref[...] = maskv

    o_ref[...] = x_ref[...] * mvec_ref[...]


def kernel(x, weights):
    w2 = weights.reshape(_R, _C)
    x3 = x.reshape(_B, _R, _C)
    mask2, sel3 = pl.pallas_call(
        _body,
        grid=(_B // _ROWS,),
        in_specs=[
            pl.BlockSpec((_R, _C), lambda i: (0, 0)),
            pl.BlockSpec((_ROWS, _R, _C), lambda i: (i, 0, 0)),
        ],
        out_specs=[
            pl.BlockSpec((_R, _C), lambda i: (0, 0)),
            pl.BlockSpec((_ROWS, _R, _C), lambda i: (i, 0, 0)),
        ],
        out_shape=[
            jax.ShapeDtypeStruct((_R, _C), jnp.float32),
            jax.ShapeDtypeStruct((_B, _R, _C), jnp.float32),
        ],
        scratch_shapes=[pltpu.VMEM((_R, _C), jnp.float32)],
    )(w2, x3)
    return (sel3.reshape(_B, _N), mask2.reshape(_N))
